# baseline (speedup 1.0000x reference)
import jax
import jax.numpy as jnp
from jax.experimental import pallas as pl
from jax.experimental.pallas import tpu as pltpu

BS = 1024


def _make_body(npos):
    def _add_body(in_ref, pos_ref, out_ref):
        i = pl.program_id(0)
        off = (i % npos) * BS
        out_ref[...] = in_ref[...] + pos_ref[pl.ds(off, BS), :]

    return _add_body


def kernel(inputs, pos_table):
    B, S, D = inputs.shape
    flat = inputs.astype(jnp.float32).reshape(B * S, D)
    npos = S // BS
    out = pl.pallas_call(
        _make_body(npos),
        grid=(B * S // BS,),
        in_specs=[
            pl.BlockSpec((BS, D), lambda i: (i, 0)),
            pl.BlockSpec((S, D), lambda i: (0, 0)),
        ],
        out_specs=pl.BlockSpec((BS, D), lambda i: (i, 0)),
        out_shape=jax.ShapeDtypeStruct((B * S, D), jnp.float32),
        compiler_params=pltpu.CompilerParams(
            dimension_semantics=("arbitrary",),
        ),
    )(flat, pos_table)
    return out.reshape(B, S, D)


# speedup vs baseline: 1.0622x; 1.0622x over previous
import jax
import jax.numpy as jnp
from jax.experimental import pallas as pl
from jax.experimental.pallas import tpu as pltpu

BS = 2048


def _make_body(npos):
    def _add_body(in_ref, pos_ref, out_ref):
        i = pl.program_id(0)
        off = (i % npos) * BS
        out_ref[...] = in_ref[...] + pos_ref[pl.ds(off, BS), :]

    return _add_body


def kernel(inputs, pos_table):
    B, S, D = inputs.shape
    flat = inputs.astype(jnp.float32).reshape(B * S, D)
    npos = S // BS
    out = pl.pallas_call(
        _make_body(npos),
        grid=(B * S // BS,),
        in_specs=[
            pl.BlockSpec((BS, D), lambda i: (i, 0)),
            pl.BlockSpec((S, D), lambda i: (0, 0)),
        ],
        out_specs=pl.BlockSpec((BS, D), lambda i: (i, 0)),
        out_shape=jax.ShapeDtypeStruct((B * S, D), jnp.float32),
        compiler_params=pltpu.CompilerParams(
            dimension_semantics=("arbitrary",),
        ),
    )(flat, pos_table)
    return out.reshape(B, S, D)
